# baseline (speedup 1.0000x reference)
import functools

import jax
import jax.numpy as jnp
from jax import lax
from jax.experimental import pallas as pl
from jax.experimental.pallas import tpu as pltpu
from jax.experimental.pallas import tpu_sc as plsc

N = 10000
NPAD = 10112
E = 320000
NW = 32
BLK = 128
NBLK = 80
NBLK0 = 32
NBLK1 = 128
NBUF = 8
EPW = NBLK * BLK
EPAD = NW * EPW
TBLK = EPAD // BLK
F = 16
ROWS_PT = NPAD // 16

_mesh = plsc.VectorSubcoreMesh(core_axis_name="c", subcore_axis_name="s")


@functools.partial(
    pl.kernel,
    out_type=jax.ShapeDtypeStruct((NW, NPAD), jnp.float32),
    mesh=_mesh,
    scratch_types=[
        pltpu.VMEM((EPW // 16, 16), jnp.int32),
        pltpu.VMEM((NPAD,), jnp.float32),
    ],
    compiler_params=pltpu.CompilerParams(needs_layout_passes=False),
)
def _deg_kernel(dst_hbm, deg_out, dst_v, deg_v):
    cid = lax.axis_index("c")
    sid = lax.axis_index("s")
    wid = sid * 2 + cid
    pltpu.sync_copy(dst_hbm.at[wid], dst_v)
    zeros = jnp.zeros((16,), jnp.float32)

    def zbody(i, carry):
        deg_v[pl.ds(i * 16, 16)] = zeros
        return carry

    lax.fori_loop(0, NPAD // 16, zbody, 0)
    ones = jnp.ones((16,), jnp.float32)

    def body(j, carry):
        plsc.addupdate_scatter(deg_v, [dst_v[j]], ones)
        return carry

    lax.fori_loop(0, EPW // 16, body, 0)
    pltpu.sync_copy(deg_v, deg_out.at[wid])


@functools.partial(
    pl.kernel,
    out_type=jax.ShapeDtypeStruct((2, NPAD, F), jnp.float32),
    mesh=_mesh,
    scratch_types=[
        pltpu.VMEM((NBLK1, BLK), jnp.int32),
        pltpu.VMEM((NBLK1, BLK), jnp.int32),
        [pltpu.VMEM((BLK, F), jnp.float32) for _ in range(NBUF)],
        pltpu.VMEM_SHARED((NPAD, F), jnp.float32),
        [pltpu.SemaphoreType.DMA for _ in range(NBUF)],
    ],
    compiler_params=pltpu.CompilerParams(
        needs_layout_passes=False, use_tc_tiling_on_sc=False
    ),
)
def _agg_kernel(src_hbm, dst_hbm, ms_hbm, out_hbm, src_v, dst_v, bufs, acc_sh, sems):
    cid = lax.axis_index("c")
    sid = lax.axis_index("s")
    base = jnp.where(cid == 0, sid * NBLK0, 16 * NBLK0 + sid * NBLK1)
    cnt = jnp.where(cid == 0, NBLK0, NBLK1)

    @pl.when(cid == 0)
    def _():
        pltpu.sync_copy(src_hbm.at[pl.ds(base, NBLK0)], src_v.at[pl.ds(0, NBLK0)])
        pltpu.sync_copy(dst_hbm.at[pl.ds(base, NBLK0)], dst_v.at[pl.ds(0, NBLK0)])

    @pl.when(cid != 0)
    def _():
        pltpu.sync_copy(src_hbm.at[pl.ds(base, NBLK1)], src_v)
        pltpu.sync_copy(dst_hbm.at[pl.ds(base, NBLK1)], dst_v)

    r0 = sid * ROWS_PT
    pltpu.sync_copy(ms_hbm.at[pl.ds(r0, ROWS_PT)], acc_sh.at[pl.ds(r0, ROWS_PT)])
    plsc.subcore_barrier()

    for b in range(NBUF):
        pltpu.async_copy(ms_hbm.at[src_v.at[b]], bufs[b], sems[b])

    def body(r, carry):
        for b in range(NBUF):
            j = r * NBUF + b
            pltpu.make_async_copy(ms_hbm.at[src_v.at[j]], bufs[b], sems[b]).wait()
            pltpu.sync_copy(bufs[b], acc_sh.at[dst_v.at[j]], add=True)
            pltpu.async_copy(ms_hbm.at[src_v.at[j + NBUF]], bufs[b], sems[b])
        return carry

    lax.fori_loop(0, cnt // NBUF - 1, body, 0)
    for b in range(NBUF):
        j = cnt - NBUF + b
        pltpu.make_async_copy(ms_hbm.at[src_v.at[j]], bufs[b], sems[b]).wait()
        pltpu.sync_copy(bufs[b], acc_sh.at[dst_v.at[j]], add=True)

    plsc.subcore_barrier()
    pltpu.sync_copy(acc_sh.at[pl.ds(r0, ROWS_PT)], out_hbm.at[cid, pl.ds(r0, ROWS_PT)])


def _tc1_body(degp_ref, x_ref, w1_ref, dinv_ref, ms1_ref):
    deg = jnp.sum(degp_ref[...], axis=1, keepdims=True) + 1.0
    dinv = lax.rsqrt(deg)
    row = lax.broadcasted_iota(jnp.int32, (NPAD, 1), 0)
    dinv = jnp.where(row < N, dinv, 0.0)
    dinv_ref[...] = dinv
    h = jnp.dot(x_ref[...], w1_ref[...], preferred_element_type=jnp.float32)
    ms1_ref[...] = h * dinv


_tc1 = pl.pallas_call(
    _tc1_body,
    out_shape=[
        jax.ShapeDtypeStruct((NPAD, 1), jnp.float32),
        jax.ShapeDtypeStruct((NPAD, F), jnp.float32),
    ],
)


def _tcmid_body(parts_ref, ms_ref, dinv_ref, b_ref, w_ref, out_ref):
    dinv = dinv_ref[...]
    agg = parts_ref[0] + parts_ref[1] - ms_ref[...]
    o = agg * dinv + b_ref[...]
    h = jnp.maximum(o, 0.0)
    out_ref[...] = jnp.dot(h, w_ref[...], preferred_element_type=jnp.float32) * dinv


_tcmid = pl.pallas_call(
    _tcmid_body,
    out_shape=jax.ShapeDtypeStruct((NPAD, F), jnp.float32),
)


def _tc4_body(parts_ref, ms_ref, dinv_ref, b_ref, out_ref):
    agg = parts_ref[0] + parts_ref[1] - ms_ref[...]
    out_ref[...] = agg * dinv_ref[...] + b_ref[...]


_tc4 = pl.pallas_call(
    _tc4_body,
    out_shape=jax.ShapeDtypeStruct((NPAD, F), jnp.float32),
)


def _padw(w, b):
    wp = jnp.zeros((F, F), jnp.float32).at[: w.shape[0], : w.shape[1]].set(w)
    bp = jnp.zeros((1, F), jnp.float32).at[0, : b.shape[0]].set(b)
    return wp, bp


def kernel(x, edge_index, W1, b1, W2, b2, W3, b3):
    ei = edge_index.astype(jnp.int32)
    npad_e = EPAD - E
    src_p = jnp.concatenate([ei[0], jnp.zeros((npad_e,), jnp.int32)])
    dst_p = jnp.concatenate(
        [ei[1], N + (jnp.arange(npad_e, dtype=jnp.int32) % (NPAD - N))]
    )
    src_blocks = src_p.reshape(TBLK, BLK)
    dst_blocks = dst_p.reshape(TBLK, BLK)
    dst_deg = dst_p.reshape(NW, EPW // 16, 16)
    x_pad = jnp.pad(x, ((0, NPAD - N), (0, 0)))
    W2p, b1p = _padw(W2, b1)
    W3p, b2p = _padw(W3, b2)
    b3p = jnp.zeros((1, F), jnp.float32).at[0, : b3.shape[0]].set(b3)

    deg_parts = _deg_kernel(dst_deg)
    dinv, ms1 = _tc1(deg_parts.T, x_pad, W1)
    parts1 = _agg_kernel(src_blocks, dst_blocks, ms1)
    ms2 = _tcmid(parts1, ms1, dinv, b1p, W2p)
    parts2 = _agg_kernel(src_blocks, dst_blocks, ms2)
    ms3 = _tcmid(parts2, ms2, dinv, b2p, W3p)
    parts3 = _agg_kernel(src_blocks, dst_blocks, ms3)
    out = _tc4(parts3, ms3, dinv, b3p)
    return out[:N, :4]


# speedup vs baseline: 1.0520x; 1.0520x over previous
import functools

import jax
import jax.numpy as jnp
from jax import lax
from jax.experimental import pallas as pl
from jax.experimental.pallas import tpu as pltpu
from jax.experimental.pallas import tpu_sc as plsc

N = 10000
NPAD = 10112
E = 320000
NW = 32
BLK = 128
NBLK = 80
NBUF = 8
EPW = NBLK * BLK
EPAD = NW * EPW
TBLK = EPAD // BLK
NBLK_W = TBLK // 16
F = 16
ROWS_PT = NPAD // 16

_mesh = plsc.VectorSubcoreMesh(core_axis_name="c", subcore_axis_name="s")


@functools.partial(
    pl.kernel,
    out_type=jax.ShapeDtypeStruct((NW, NPAD), jnp.float32),
    mesh=_mesh,
    scratch_types=[
        pltpu.VMEM((EPW // 16, 16), jnp.int32),
        pltpu.VMEM((NPAD,), jnp.float32),
    ],
    compiler_params=pltpu.CompilerParams(needs_layout_passes=False),
)
def _deg_kernel(dst_hbm, deg_out, dst_v, deg_v):
    cid = lax.axis_index("c")
    sid = lax.axis_index("s")
    wid = sid * 2 + cid
    pltpu.sync_copy(dst_hbm.at[wid], dst_v)
    zeros = jnp.zeros((16,), jnp.float32)

    def zbody(i, carry):
        deg_v[pl.ds(i * 16, 16)] = zeros
        return carry

    lax.fori_loop(0, NPAD // 16, zbody, 0)
    ones = jnp.ones((16,), jnp.float32)

    def body(j, carry):
        plsc.addupdate_scatter(deg_v, [dst_v[j]], ones)
        return carry

    lax.fori_loop(0, EPW // 16, body, 0)
    pltpu.sync_copy(deg_v, deg_out.at[wid])


@functools.partial(
    pl.kernel,
    out_type=jax.ShapeDtypeStruct((NPAD, F), jnp.float32),
    mesh=_mesh,
    scratch_types=[
        pltpu.VMEM((NBLK_W, BLK), jnp.int32),
        pltpu.VMEM((NBLK_W, BLK), jnp.int32),
        [pltpu.VMEM((BLK, F), jnp.float32) for _ in range(NBUF)],
        pltpu.VMEM_SHARED((NPAD, F), jnp.float32),
        [pltpu.SemaphoreType.DMA for _ in range(NBUF)],
    ],
    compiler_params=pltpu.CompilerParams(
        needs_layout_passes=False, use_tc_tiling_on_sc=False
    ),
)
def _agg_kernel(src_hbm, dst_hbm, ms_hbm, out_hbm, src_v, dst_v, bufs, acc_sh, sems):
    cid = lax.axis_index("c")
    sid = lax.axis_index("s")

    @pl.when(cid == 0)
    def _():
        base = sid * NBLK_W
        pltpu.sync_copy(src_hbm.at[pl.ds(base, NBLK_W)], src_v)
        pltpu.sync_copy(dst_hbm.at[pl.ds(base, NBLK_W)], dst_v)
        r0 = sid * ROWS_PT
        pltpu.sync_copy(ms_hbm.at[pl.ds(r0, ROWS_PT)], acc_sh.at[pl.ds(r0, ROWS_PT)])
        plsc.subcore_barrier()

        for b in range(NBUF):
            pltpu.async_copy(ms_hbm.at[src_v.at[b]], bufs[b], sems[b])

        def body(r, carry):
            for b in range(NBUF):
                j = r * NBUF + b
                pltpu.make_async_copy(ms_hbm.at[src_v.at[j]], bufs[b], sems[b]).wait()
                pltpu.sync_copy(bufs[b], acc_sh.at[dst_v.at[j]], add=True)
                pltpu.async_copy(ms_hbm.at[src_v.at[j + NBUF]], bufs[b], sems[b])
            return carry

        lax.fori_loop(0, NBLK_W // NBUF - 1, body, 0)
        for b in range(NBUF):
            j = NBLK_W - NBUF + b
            pltpu.make_async_copy(ms_hbm.at[src_v.at[j]], bufs[b], sems[b]).wait()
            pltpu.sync_copy(bufs[b], acc_sh.at[dst_v.at[j]], add=True)

        plsc.subcore_barrier()
        pltpu.sync_copy(acc_sh.at[pl.ds(r0, ROWS_PT)], out_hbm.at[pl.ds(r0, ROWS_PT)])


def _tc1_body(degp_ref, x_ref, w1_ref, dinv_ref, ms1_ref):
    deg = jnp.sum(degp_ref[...], axis=1, keepdims=True) + 1.0
    dinv = lax.rsqrt(deg)
    row = lax.broadcasted_iota(jnp.int32, (NPAD, 1), 0)
    dinv = jnp.where(row < N, dinv, 0.0)
    dinv_ref[...] = dinv
    h = jnp.dot(x_ref[...], w1_ref[...], preferred_element_type=jnp.float32)
    ms1_ref[...] = h * dinv


_tc1 = pl.pallas_call(
    _tc1_body,
    out_shape=[
        jax.ShapeDtypeStruct((NPAD, 1), jnp.float32),
        jax.ShapeDtypeStruct((NPAD, F), jnp.float32),
    ],
)


def _tcmid_body(parts_ref, dinv_ref, b_ref, w_ref, out_ref):
    dinv = dinv_ref[...]
    o = parts_ref[...] * dinv + b_ref[...]
    h = jnp.maximum(o, 0.0)
    out_ref[...] = jnp.dot(h, w_ref[...], preferred_element_type=jnp.float32) * dinv


_tcmid = pl.pallas_call(
    _tcmid_body,
    out_shape=jax.ShapeDtypeStruct((NPAD, F), jnp.float32),
)


def _tc4_body(parts_ref, dinv_ref, b_ref, out_ref):
    out_ref[...] = parts_ref[...] * dinv_ref[...] + b_ref[...]


_tc4 = pl.pallas_call(
    _tc4_body,
    out_shape=jax.ShapeDtypeStruct((NPAD, F), jnp.float32),
)


def _padw(w, b):
    wp = jnp.zeros((F, F), jnp.float32).at[: w.shape[0], : w.shape[1]].set(w)
    bp = jnp.zeros((1, F), jnp.float32).at[0, : b.shape[0]].set(b)
    return wp, bp


def kernel(x, edge_index, W1, b1, W2, b2, W3, b3):
    ei = edge_index.astype(jnp.int32)
    npad_e = EPAD - E
    src_p = jnp.concatenate([ei[0], jnp.zeros((npad_e,), jnp.int32)])
    dst_p = jnp.concatenate(
        [ei[1], N + (jnp.arange(npad_e, dtype=jnp.int32) % (NPAD - N))]
    )
    src_blocks = src_p.reshape(TBLK, BLK)
    dst_blocks = dst_p.reshape(TBLK, BLK)
    dst_deg = dst_p.reshape(NW, EPW // 16, 16)
    x_pad = jnp.pad(x, ((0, NPAD - N), (0, 0)))
    W2p, b1p = _padw(W2, b1)
    W3p, b2p = _padw(W3, b2)
    b3p = jnp.zeros((1, F), jnp.float32).at[0, : b3.shape[0]].set(b3)

    deg_parts = _deg_kernel(dst_deg)
    dinv, ms1 = _tc1(deg_parts.T, x_pad, W1)
    parts1 = _agg_kernel(src_blocks, dst_blocks, ms1)
    ms2 = _tcmid(parts1, dinv, b1p, W2p)
    parts2 = _agg_kernel(src_blocks, dst_blocks, ms2)
    ms3 = _tcmid(parts2, dinv, b2p, W3p)
    parts3 = _agg_kernel(src_blocks, dst_blocks, ms3)
    out = _tc4(parts3, dinv, b3p)
    return out[:N, :4]


# speedup vs baseline: 1.3666x; 1.2991x over previous
import functools

import jax
import jax.numpy as jnp
from jax import lax
from jax.experimental import pallas as pl
from jax.experimental.pallas import tpu as pltpu
from jax.experimental.pallas import tpu_sc as plsc

N = 10000
NPAD = 10112
E = 320000
NW = 32
BLK = 128
NBUF = 8
EPW = 10240
EPAD = NW * EPW
TBLK = EPAD // BLK
NBLK_W = TBLK // 16
ROWS_PT = NPAD // 16

_mesh = plsc.VectorSubcoreMesh(core_axis_name="c", subcore_axis_name="s")


@functools.partial(
    pl.kernel,
    out_type=jax.ShapeDtypeStruct((NW, NPAD), jnp.float32),
    mesh=_mesh,
    scratch_types=[
        pltpu.VMEM((EPW // 16, 16), jnp.int32),
        pltpu.VMEM((NPAD,), jnp.float32),
    ],
    compiler_params=pltpu.CompilerParams(needs_layout_passes=False),
)
def _deg_kernel(dst_hbm, deg_out, dst_v, deg_v):
    cid = lax.axis_index("c")
    sid = lax.axis_index("s")
    wid = sid * 2 + cid
    pltpu.sync_copy(dst_hbm.at[wid], dst_v)
    zeros = jnp.zeros((16,), jnp.float32)

    def zbody(i, carry):
        deg_v[pl.ds(i * 16, 16)] = zeros
        return carry

    lax.fori_loop(0, NPAD // 16, zbody, 0)
    ones = jnp.ones((16,), jnp.float32)

    def body(j, carry):
        plsc.addupdate_scatter(deg_v, [dst_v[j]], ones)
        return carry

    lax.fori_loop(0, EPW // 16, body, 0)
    pltpu.sync_copy(deg_v, deg_out.at[wid])


def _make_agg(f):
    @functools.partial(
        pl.kernel,
        out_type=jax.ShapeDtypeStruct((NPAD, f), jnp.float32),
        mesh=_mesh,
        scratch_types=[
            pltpu.VMEM((NBLK_W, BLK), jnp.int32),
            pltpu.VMEM((NBLK_W, BLK), jnp.int32),
            [pltpu.VMEM((BLK, f), jnp.float32) for _ in range(NBUF)],
            pltpu.VMEM_SHARED((NPAD, f), jnp.float32),
            pltpu.VMEM_SHARED((NPAD, f), jnp.float32),
            [pltpu.SemaphoreType.DMA for _ in range(NBUF)],
        ],
        compiler_params=pltpu.CompilerParams(
            needs_layout_passes=False, use_tc_tiling_on_sc=False
        ),
    )
    def _agg(src_hbm, dst_hbm, ms_hbm, out_hbm, src_v, dst_v, bufs, tbl_sh, acc_sh, sems):
        cid = lax.axis_index("c")
        sid = lax.axis_index("s")

        @pl.when(cid == 0)
        def _():
            base = sid * NBLK_W
            pltpu.sync_copy(src_hbm.at[pl.ds(base, NBLK_W)], src_v)
            pltpu.sync_copy(dst_hbm.at[pl.ds(base, NBLK_W)], dst_v)
            r0 = sid * ROWS_PT
            pltpu.sync_copy(ms_hbm.at[pl.ds(r0, ROWS_PT)], tbl_sh.at[pl.ds(r0, ROWS_PT)])
            pltpu.sync_copy(ms_hbm.at[pl.ds(r0, ROWS_PT)], acc_sh.at[pl.ds(r0, ROWS_PT)])
            plsc.subcore_barrier()

            for b in range(NBUF):
                pltpu.async_copy(tbl_sh.at[src_v.at[b]], bufs[b], sems[b])

            def body(r, carry):
                for b in range(NBUF):
                    j = r * NBUF + b
                    pltpu.make_async_copy(tbl_sh.at[src_v.at[j]], bufs[b], sems[b]).wait()
                    pltpu.sync_copy(bufs[b], acc_sh.at[dst_v.at[j]], add=True)
                    pltpu.async_copy(tbl_sh.at[src_v.at[j + NBUF]], bufs[b], sems[b])
                return carry

            lax.fori_loop(0, NBLK_W // NBUF - 1, body, 0)
            for b in range(NBUF):
                j = NBLK_W - NBUF + b
                pltpu.make_async_copy(tbl_sh.at[src_v.at[j]], bufs[b], sems[b]).wait()
                pltpu.sync_copy(bufs[b], acc_sh.at[dst_v.at[j]], add=True)

            plsc.subcore_barrier()
            pltpu.sync_copy(acc_sh.at[pl.ds(r0, ROWS_PT)], out_hbm.at[pl.ds(r0, ROWS_PT)])

    return _agg


_agg16 = _make_agg(16)


def _tc1_body(degp_ref, x_ref, w1_ref, dinv_ref, ms1_ref):
    deg = jnp.sum(degp_ref[...], axis=1, keepdims=True) + 1.0
    dinv = lax.rsqrt(deg)
    row = lax.broadcasted_iota(jnp.int32, (NPAD, 1), 0)
    dinv = jnp.where(row < N, dinv, 0.0)
    dinv_ref[...] = dinv
    h = jnp.dot(x_ref[...], w1_ref[...], preferred_element_type=jnp.float32)
    ms1_ref[...] = h * dinv


_tc1 = pl.pallas_call(
    _tc1_body,
    out_shape=[
        jax.ShapeDtypeStruct((NPAD, 1), jnp.float32),
        jax.ShapeDtypeStruct((NPAD, 16), jnp.float32),
    ],
)


def _tcmid_body(parts_ref, dinv_ref, b_ref, w_ref, out_ref):
    dinv = dinv_ref[...]
    o = parts_ref[...] * dinv + b_ref[...]
    h = jnp.maximum(o, 0.0)
    out_ref[...] = jnp.dot(h, w_ref[...], preferred_element_type=jnp.float32) * dinv


def _make_tcmid(fout):
    return pl.pallas_call(
        _tcmid_body,
        out_shape=jax.ShapeDtypeStruct((NPAD, fout), jnp.float32),
    )


_tcmid16 = _make_tcmid(16)


def _tc4_body(parts_ref, dinv_ref, b_ref, out_ref):
    out_ref[...] = parts_ref[...] * dinv_ref[...] + b_ref[...]


_tc4 = pl.pallas_call(
    _tc4_body,
    out_shape=jax.ShapeDtypeStruct((NPAD, 16), jnp.float32),
)


def _padw(w, b):
    wp = jnp.zeros((16, 16), jnp.float32).at[: w.shape[0], : w.shape[1]].set(w)
    bp = jnp.zeros((1, 16), jnp.float32).at[0, : b.shape[0]].set(b)
    return wp, bp


def kernel(x, edge_index, W1, b1, W2, b2, W3, b3):
    ei = edge_index.astype(jnp.int32)
    npad_e = EPAD - E
    src_p = jnp.concatenate([ei[0], jnp.zeros((npad_e,), jnp.int32)])
    dst_p = jnp.concatenate(
        [ei[1], N + (jnp.arange(npad_e, dtype=jnp.int32) % (NPAD - N))]
    )
    src_blocks = src_p.reshape(TBLK, BLK)
    dst_blocks = dst_p.reshape(TBLK, BLK)
    dst_deg = dst_p.reshape(NW, EPW // 16, 16)
    x_pad = jnp.pad(x, ((0, NPAD - N), (0, 0)))

    W2p, b1p = _padw(W2, b1)
    W3p, b2p = _padw(W3, b2)
    b3p = jnp.zeros((1, 16), jnp.float32).at[0, : b3.shape[0]].set(b3)

    deg_parts = _deg_kernel(dst_deg)
    dinv, ms1 = _tc1(deg_parts.T, x_pad, W1)
    parts1 = _agg16(src_blocks, dst_blocks, ms1)
    ms2 = _tcmid16(parts1, dinv, b1p, W2p)
    parts2 = _agg16(src_blocks, dst_blocks, ms2)
    ms3 = _tcmid16(parts2, dinv, b2p, W3p)
    parts3 = _agg16(src_blocks, dst_blocks, ms3)
    out = _tc4(parts3, dinv, b3p)
    return out[:N, :4]


# speedup vs baseline: 1.5898x; 1.1633x over previous
import functools

import jax
import jax.numpy as jnp
from jax import lax
from jax.experimental import pallas as pl
from jax.experimental.pallas import tpu as pltpu
from jax.experimental.pallas import tpu_sc as plsc

N = 10000
NPAD = 10112
E = 320000
NW = 32
BLK = 128
NBUF = 8
EPW = 10240
EPAD = NW * EPW
TBLK = EPAD // BLK
NBLK_W = TBLK // 16
ROWS_PT = NPAD // 16

_mesh = plsc.VectorSubcoreMesh(core_axis_name="c", subcore_axis_name="s")


@functools.partial(
    pl.kernel,
    out_type=jax.ShapeDtypeStruct((NW, NPAD), jnp.float32),
    mesh=_mesh,
    scratch_types=[
        pltpu.VMEM((EPW // 16, 16), jnp.int32),
        pltpu.VMEM((NPAD,), jnp.float32),
    ],
    compiler_params=pltpu.CompilerParams(needs_layout_passes=False),
)
def _deg_kernel(dst_hbm, deg_out, dst_v, deg_v):
    cid = lax.axis_index("c")
    sid = lax.axis_index("s")
    wid = sid * 2 + cid
    pltpu.sync_copy(dst_hbm.at[wid], dst_v)
    zeros = jnp.zeros((16,), jnp.float32)

    def zbody(i, carry):
        deg_v[pl.ds(i * 16, 16)] = zeros
        return carry

    lax.fori_loop(0, NPAD // 16, zbody, 0)
    ones = jnp.ones((16,), jnp.float32)

    def body(j, carry):
        plsc.addupdate_scatter(deg_v, [dst_v[j]], ones)
        return carry

    lax.fori_loop(0, EPW // 16, body, 0)
    pltpu.sync_copy(deg_v, deg_out.at[wid])


def _make_agg(f):
    nblk_w = TBLK // NW

    @functools.partial(
        pl.kernel,
        out_type=jax.ShapeDtypeStruct((2, NPAD, f), jnp.float32),
        mesh=_mesh,
        scratch_types=[
            pltpu.VMEM((TBLK // NW, BLK), jnp.int32),
            pltpu.VMEM((TBLK // NW, BLK), jnp.int32),
            [pltpu.VMEM((BLK, f), jnp.float32) for _ in range(NBUF)],
            pltpu.VMEM_SHARED((NPAD, f), jnp.float32),
            pltpu.VMEM_SHARED((NPAD, f), jnp.float32),
            [pltpu.SemaphoreType.DMA for _ in range(NBUF)],
        ],
        compiler_params=pltpu.CompilerParams(
            needs_layout_passes=False, use_tc_tiling_on_sc=False
        ),
    )
    def _agg(src_hbm, dst_hbm, ms_hbm, out_hbm, src_v, dst_v, bufs, tbl_sh, acc_sh, sems):
        cid = lax.axis_index("c")
        sid = lax.axis_index("s")
        base = (sid * 2 + cid) * nblk_w
        pltpu.sync_copy(src_hbm.at[pl.ds(base, nblk_w)], src_v)
        pltpu.sync_copy(dst_hbm.at[pl.ds(base, nblk_w)], dst_v)
        r0 = sid * ROWS_PT
        pltpu.sync_copy(ms_hbm.at[pl.ds(r0, ROWS_PT)], tbl_sh.at[pl.ds(r0, ROWS_PT)])
        pltpu.sync_copy(ms_hbm.at[pl.ds(r0, ROWS_PT)], acc_sh.at[pl.ds(r0, ROWS_PT)])
        plsc.subcore_barrier()

        for b in range(NBUF):
            pltpu.async_copy(tbl_sh.at[src_v.at[b]], bufs[b], sems[b])

        def body(r, carry):
            for b in range(NBUF):
                j = r * NBUF + b
                pltpu.make_async_copy(tbl_sh.at[src_v.at[j]], bufs[b], sems[b]).wait()
                pltpu.sync_copy(bufs[b], acc_sh.at[dst_v.at[j]], add=True)
                pltpu.async_copy(tbl_sh.at[src_v.at[j + NBUF]], bufs[b], sems[b])
            return carry

        lax.fori_loop(0, nblk_w // NBUF - 1, body, 0)
        for b in range(NBUF):
            j = nblk_w - NBUF + b
            pltpu.make_async_copy(tbl_sh.at[src_v.at[j]], bufs[b], sems[b]).wait()
            pltpu.sync_copy(bufs[b], acc_sh.at[dst_v.at[j]], add=True)

        plsc.subcore_barrier()
        pltpu.sync_copy(acc_sh.at[pl.ds(r0, ROWS_PT)], out_hbm.at[cid, pl.ds(r0, ROWS_PT)])

    return _agg


_agg16 = _make_agg(16)


def _tc1_body(degp_ref, x_ref, w1_ref, dinv_ref, ms1_ref):
    deg = jnp.sum(degp_ref[...], axis=1, keepdims=True) + 1.0
    dinv = lax.rsqrt(deg)
    row = lax.broadcasted_iota(jnp.int32, (NPAD, 1), 0)
    dinv = jnp.where(row < N, dinv, 0.0)
    dinv_ref[...] = dinv
    h = jnp.dot(x_ref[...], w1_ref[...], preferred_element_type=jnp.float32)
    ms1_ref[...] = h * dinv


_tc1 = pl.pallas_call(
    _tc1_body,
    out_shape=[
        jax.ShapeDtypeStruct((NPAD, 1), jnp.float32),
        jax.ShapeDtypeStruct((NPAD, 16), jnp.float32),
    ],
)


def _tcmid_body(parts_ref, ms_ref, dinv_ref, b_ref, w_ref, out_ref):
    dinv = dinv_ref[...]
    agg = parts_ref[0] + parts_ref[1] - ms_ref[...]
    o = agg * dinv + b_ref[...]
    h = jnp.maximum(o, 0.0)
    out_ref[...] = jnp.dot(h, w_ref[...], preferred_element_type=jnp.float32) * dinv


def _make_tcmid(fout):
    return pl.pallas_call(
        _tcmid_body,
        out_shape=jax.ShapeDtypeStruct((NPAD, fout), jnp.float32),
    )


_tcmid16 = _make_tcmid(16)


def _tc4_body(parts_ref, ms_ref, dinv_ref, b_ref, out_ref):
    agg = parts_ref[0] + parts_ref[1] - ms_ref[...]
    out_ref[...] = agg * dinv_ref[...] + b_ref[...]


_tc4 = pl.pallas_call(
    _tc4_body,
    out_shape=jax.ShapeDtypeStruct((NPAD, 16), jnp.float32),
)


def _padw(w, b):
    wp = jnp.zeros((16, 16), jnp.float32).at[: w.shape[0], : w.shape[1]].set(w)
    bp = jnp.zeros((1, 16), jnp.float32).at[0, : b.shape[0]].set(b)
    return wp, bp


def kernel(x, edge_index, W1, b1, W2, b2, W3, b3):
    ei = edge_index.astype(jnp.int32)
    npad_e = EPAD - E
    src_p = jnp.concatenate([ei[0], jnp.zeros((npad_e,), jnp.int32)])
    dst_p = jnp.concatenate(
        [ei[1], N + (jnp.arange(npad_e, dtype=jnp.int32) % (NPAD - N))]
    )
    src_blocks = src_p.reshape(TBLK, BLK)
    dst_blocks = dst_p.reshape(TBLK, BLK)
    dst_deg = dst_p.reshape(NW, EPW // 16, 16)
    x_pad = jnp.pad(x, ((0, NPAD - N), (0, 0)))

    W2p, b1p = _padw(W2, b1)
    W3p, b2p = _padw(W3, b2)
    b3p = jnp.zeros((1, 16), jnp.float32).at[0, : b3.shape[0]].set(b3)

    deg_parts = _deg_kernel(dst_deg)
    dinv, ms1 = _tc1(deg_parts.T, x_pad, W1)
    parts1 = _agg16(src_blocks, dst_blocks, ms1)
    ms2 = _tcmid16(parts1, ms1, dinv, b1p, W2p)
    parts2 = _agg16(src_blocks, dst_blocks, ms2)
    ms3 = _tcmid16(parts2, ms2, dinv, b2p, W3p)
    parts3 = _agg16(src_blocks, dst_blocks, ms3)
    out = _tc4(parts3, ms3, dinv, b3p)
    return out[:N, :4]
